# baseline (speedup 1.0000x reference)
import jax
import jax.numpy as jnp
from jax.experimental import pallas as pl

_N_RAYS = 4096
_N_SAMPLES = 192
_NEAR, _FAR = 2.0, 6.0
_STEP = (_FAR - _NEAR) / _N_SAMPLES
_LOG_EPS = -23.025850929940457
_BLOCK_R = 1024
_N_CHAINS = 4


def _render_rays(rays, w6, b6, wcat, bs, br, tri, t_mid2):
    f32 = jnp.float32
    r = rays.shape[0]
    ab = jnp.dot(rays, w6, preferred_element_type=f32) + b6
    c = ab.reshape(r, 2, 128)
    ones = jnp.ones((1, _N_SAMPLES), dtype=f32)
    tmat = jnp.concatenate([ones[:, None, :], t_mid2[:, None, :]], axis=1)
    tmat = jnp.broadcast_to(tmat, (r, 2, _N_SAMPLES))
    h = jax.lax.dot_general(
        c, tmat, dimension_numbers=(((1,), (1,)), ((0,), (0,))),
        preferred_element_type=f32)
    h = jnp.maximum(h, 0.0)

    wcat_b = jnp.broadcast_to(wcat[None], (r, 4, 128))
    z = jax.lax.dot_general(
        wcat_b, h, dimension_numbers=(((2,), (1,)), ((0,), (0,))),
        preferred_element_type=f32)

    sigma = jax.nn.softplus(z[:, 0, :] + bs)
    x = sigma * _STEP
    alpha = 1.0 - jnp.exp(-x)
    log_trans = jnp.maximum(-x, _LOG_EPS)
    excl = jnp.dot(log_trans, tri, preferred_element_type=f32)
    weights = alpha * jnp.exp(excl)

    outs = []
    for ch in range(3):
        rgb_c = jax.nn.sigmoid(z[:, 1 + ch, :] + br[ch])
        outs.append(jnp.sum(weights * rgb_c, axis=-1)[:, None])
    rgb = jnp.concatenate(outs, axis=1)
    op = jnp.sum(weights, axis=-1)[:, None]
    depth = jnp.sum(weights * t_mid2, axis=-1)[:, None]
    return rgb, op, depth


def _render_block(rays_ref, w6_ref, b6_ref, wcat_ref, bs_ref, br_ref, tri_ref,
                  rgb_ref, op_ref, depth_ref):
    f32 = jnp.float32
    w6 = w6_ref[...]
    b6 = b6_ref[...]
    wcat = wcat_ref[...]
    tri = tri_ref[...]
    bs = bs_ref[0, 0]
    br = (br_ref[0, 0], br_ref[0, 1], br_ref[0, 2])
    s_idx = jax.lax.broadcasted_iota(jnp.int32, (1, _N_SAMPLES), 1).astype(f32)
    t_mid2 = _NEAR + (s_idx + 0.5) * _STEP

    half = _BLOCK_R // _N_CHAINS
    for i in range(_N_CHAINS):
        rows = pl.ds(i * half, half)
        rays = rays_ref[rows, :] * (2.0 / 3.0)
        rgb, op, depth = _render_rays(rays, w6, b6, wcat, bs, br, tri, t_mid2)
        rgb_ref[rows, :] = rgb
        op_ref[rows, :] = op
        depth_ref[rows, :] = depth


@jax.jit
def kernel(rays, W1, b1, W_sigma, b_sigma, W_rgb, b_rgb):
    n_rays = rays.shape[0]
    wcat = jnp.concatenate([W_sigma, W_rgb], axis=1).T
    zero3 = jnp.zeros((3, 128), dtype=jnp.float32)
    w6 = jnp.concatenate(
        [jnp.concatenate([W1, zero3], axis=1),
         jnp.concatenate([zero3, W1], axis=1)], axis=0)
    b6 = jnp.concatenate([b1, jnp.zeros_like(b1)]).reshape(1, 256)
    bs_2d = b_sigma.reshape(1, 1)
    br_2d = b_rgb.reshape(1, 3)
    s = _N_SAMPLES
    tri = (jnp.arange(s, dtype=jnp.int32)[:, None]
           < jnp.arange(s, dtype=jnp.int32)[None, :]).astype(jnp.float32)
    grid = (n_rays // _BLOCK_R,)
    rgb, op, depth = pl.pallas_call(
        _render_block,
        grid=grid,
        in_specs=[
            pl.BlockSpec((_BLOCK_R, 6), lambda i: (i, 0)),
            pl.BlockSpec((6, 256), lambda i: (0, 0)),
            pl.BlockSpec((1, 256), lambda i: (0, 0)),
            pl.BlockSpec((4, 128), lambda i: (0, 0)),
            pl.BlockSpec((1, 1), lambda i: (0, 0)),
            pl.BlockSpec((1, 3), lambda i: (0, 0)),
            pl.BlockSpec((s, s), lambda i: (0, 0)),
        ],
        out_specs=[
            pl.BlockSpec((_BLOCK_R, 3), lambda i: (i, 0)),
            pl.BlockSpec((_BLOCK_R, 1), lambda i: (i, 0)),
            pl.BlockSpec((_BLOCK_R, 1), lambda i: (i, 0)),
        ],
        out_shape=[
            jax.ShapeDtypeStruct((n_rays, 3), jnp.float32),
            jax.ShapeDtypeStruct((n_rays, 1), jnp.float32),
            jax.ShapeDtypeStruct((n_rays, 1), jnp.float32),
        ],
    )(rays, w6, b6, wcat, bs_2d, br_2d, tri)
    return rgb, op[:, 0], depth[:, 0]


# speedup vs baseline: 1.3470x; 1.3470x over previous
import jax
import jax.numpy as jnp
from jax.experimental import pallas as pl

_N_RAYS = 4096
_N_SAMPLES = 192
_NEAR, _FAR = 2.0, 6.0
_STEP = (_FAR - _NEAR) / _N_SAMPLES
_LOG_EPS = -23.025850929940457
_BLOCK_P = 256
_N_CHAINS = 2


def _render_pairs(pairs, w12, b12, wbd, bs, br, tri, t_mid2):
    f32 = jnp.float32
    p = pairs.shape[0]
    ab = jnp.dot(pairs, w12, preferred_element_type=f32) + b12
    c = ab.reshape(p, 2, 256)
    ones = jnp.ones((1, _N_SAMPLES), dtype=f32)
    tmat = jnp.concatenate([ones[:, None, :], t_mid2[:, None, :]], axis=1)
    tmat = jnp.broadcast_to(tmat, (p, 2, _N_SAMPLES))
    h = jax.lax.dot_general(
        c, tmat, dimension_numbers=(((1,), (1,)), ((0,), (0,))),
        preferred_element_type=f32)
    h = jnp.maximum(h, 0.0)

    wbd_b = jnp.broadcast_to(wbd[None], (p, 8, 256))
    z = jax.lax.dot_general(
        wbd_b, h, dimension_numbers=(((2,), (1,)), ((0,), (0,))),
        preferred_element_type=f32)

    outs = []
    for e in range(2):
        sigma = jax.nn.softplus(z[:, 4 * e, :] + bs)
        x = sigma * _STEP
        alpha = 1.0 - jnp.exp(-x)
        log_trans = jnp.maximum(-x, _LOG_EPS)
        excl = jnp.dot(log_trans, tri, preferred_element_type=f32)
        weights = alpha * jnp.exp(excl)

        cols = []
        for ch in range(3):
            rgb_c = jax.nn.sigmoid(z[:, 4 * e + 1 + ch, :] + br[ch])
            cols.append(jnp.sum(weights * rgb_c, axis=-1)[:, None])
        rgb = jnp.concatenate(cols, axis=1)
        op = jnp.sum(weights, axis=-1)[:, None]
        depth = jnp.sum(weights * t_mid2, axis=-1)[:, None]
        outs.append((rgb, op, depth))
    return outs


def _render_block(pairs_ref, w12_ref, b12_ref, wbd_ref, bs_ref, br_ref, tri_ref,
                  rgb0_ref, rgb1_ref, op0_ref, op1_ref, d0_ref, d1_ref):
    f32 = jnp.float32
    w12 = w12_ref[...]
    b12 = b12_ref[...]
    wbd = wbd_ref[...]
    tri = tri_ref[...]
    bs = bs_ref[0, 0]
    br = (br_ref[0, 0], br_ref[0, 1], br_ref[0, 2])
    s_idx = jax.lax.broadcasted_iota(jnp.int32, (1, _N_SAMPLES), 1).astype(f32)
    t_mid2 = _NEAR + (s_idx + 0.5) * _STEP

    half = _BLOCK_P // _N_CHAINS
    for i in range(_N_CHAINS):
        rows = pl.ds(i * half, half)
        pairs = pairs_ref[rows, :] * (2.0 / 3.0)
        outs = _render_pairs(pairs, w12, b12, wbd, bs, br, tri, t_mid2)
        (rgb0, op0, d0), (rgb1, op1, d1) = outs
        rgb0_ref[rows, :] = rgb0
        rgb1_ref[rows, :] = rgb1
        op0_ref[rows, :] = op0
        op1_ref[rows, :] = op1
        d0_ref[rows, :] = d0
        d1_ref[rows, :] = d1


@jax.jit
def kernel(rays, W1, b1, W_sigma, b_sigma, W_rgb, b_rgb):
    n_rays = rays.shape[0]
    n_pairs = n_rays // 2
    f32 = jnp.float32
    wcat = jnp.concatenate([W_sigma, W_rgb], axis=1).T
    z128 = jnp.zeros((4, 128), dtype=f32)
    wbd = jnp.concatenate(
        [jnp.concatenate([wcat, z128], axis=1),
         jnp.concatenate([z128, wcat], axis=1)], axis=0)
    w12 = jnp.zeros((12, 512), dtype=f32)
    w12 = w12.at[0:3, 0:128].set(W1)
    w12 = w12.at[6:9, 128:256].set(W1)
    w12 = w12.at[3:6, 256:384].set(W1)
    w12 = w12.at[9:12, 384:512].set(W1)
    b12 = jnp.concatenate([b1, b1, jnp.zeros(256, dtype=f32)]).reshape(1, 512)
    bs_2d = b_sigma.reshape(1, 1)
    br_2d = b_rgb.reshape(1, 3)
    s = _N_SAMPLES
    tri = (jnp.arange(s, dtype=jnp.int32)[:, None]
           < jnp.arange(s, dtype=jnp.int32)[None, :]).astype(f32)
    pairs = rays.reshape(n_pairs, 12)
    grid = (n_pairs // _BLOCK_P,)
    out2 = [
        jax.ShapeDtypeStruct((n_pairs, 3), f32),
        jax.ShapeDtypeStruct((n_pairs, 3), f32),
        jax.ShapeDtypeStruct((n_pairs, 1), f32),
        jax.ShapeDtypeStruct((n_pairs, 1), f32),
        jax.ShapeDtypeStruct((n_pairs, 1), f32),
        jax.ShapeDtypeStruct((n_pairs, 1), f32),
    ]
    rgb0, rgb1, op0, op1, d0, d1 = pl.pallas_call(
        _render_block,
        grid=grid,
        in_specs=[
            pl.BlockSpec((_BLOCK_P, 12), lambda i: (i, 0)),
            pl.BlockSpec((12, 512), lambda i: (0, 0)),
            pl.BlockSpec((1, 512), lambda i: (0, 0)),
            pl.BlockSpec((8, 256), lambda i: (0, 0)),
            pl.BlockSpec((1, 1), lambda i: (0, 0)),
            pl.BlockSpec((1, 3), lambda i: (0, 0)),
            pl.BlockSpec((s, s), lambda i: (0, 0)),
        ],
        out_specs=[
            pl.BlockSpec((_BLOCK_P, 3), lambda i: (i, 0)),
            pl.BlockSpec((_BLOCK_P, 3), lambda i: (i, 0)),
            pl.BlockSpec((_BLOCK_P, 1), lambda i: (i, 0)),
            pl.BlockSpec((_BLOCK_P, 1), lambda i: (i, 0)),
            pl.BlockSpec((_BLOCK_P, 1), lambda i: (i, 0)),
            pl.BlockSpec((_BLOCK_P, 1), lambda i: (i, 0)),
        ],
        out_shape=out2,
    )(pairs, w12, b12, wbd, bs_2d, br_2d, tri)
    rgb = jnp.stack([rgb0, rgb1], axis=1).reshape(n_rays, 3)
    op = jnp.stack([op0[:, 0], op1[:, 0]], axis=1).reshape(n_rays)
    depth = jnp.stack([d0[:, 0], d1[:, 0]], axis=1).reshape(n_rays)
    return rgb, op, depth


# speedup vs baseline: 1.3478x; 1.0006x over previous
import jax
import jax.numpy as jnp
from jax.experimental import pallas as pl

_N_RAYS = 4096
_N_SAMPLES = 192
_NEAR, _FAR = 2.0, 6.0
_STEP = (_FAR - _NEAR) / _N_SAMPLES
_LOG_EPS = -23.025850929940457
_BLOCK_P = 256
_N_CHAINS = 2


def _render_pairs(pairs, w12, b12, wbd, bs, br, tri, t_mid2):
    f32 = jnp.float32
    p = pairs.shape[0]
    ab = jnp.dot(pairs, w12, preferred_element_type=f32) + b12
    c = ab.reshape(p, 2, 256)
    ones = jnp.ones((1, _N_SAMPLES), dtype=f32)
    tmat = jnp.concatenate([ones[:, None, :], t_mid2[:, None, :]], axis=1)
    tmat = jnp.broadcast_to(tmat, (p, 2, _N_SAMPLES))
    h = jax.lax.dot_general(
        c, tmat, dimension_numbers=(((1,), (1,)), ((0,), (0,))),
        preferred_element_type=f32)
    h = jnp.maximum(h, 0.0).astype(jnp.bfloat16)

    wbd_b = jnp.broadcast_to(wbd[None].astype(jnp.bfloat16), (p, 8, 256))
    z = jax.lax.dot_general(
        wbd_b, h, dimension_numbers=(((2,), (1,)), ((0,), (0,))),
        preferred_element_type=f32)

    outs = []
    for e in range(2):
        sigma = jax.nn.softplus(z[:, 4 * e, :] + bs)
        x = sigma * _STEP
        alpha = 1.0 - jnp.exp(-x)
        log_trans = jnp.maximum(-x, _LOG_EPS)
        excl = jnp.dot(log_trans, tri, preferred_element_type=f32)
        weights = alpha * jnp.exp(excl)

        cols = []
        for ch in range(3):
            rgb_c = jax.nn.sigmoid(z[:, 4 * e + 1 + ch, :] + br[ch])
            cols.append(jnp.sum(weights * rgb_c, axis=-1)[:, None])
        rgb = jnp.concatenate(cols, axis=1)
        op = jnp.sum(weights, axis=-1)[:, None]
        depth = jnp.sum(weights * t_mid2, axis=-1)[:, None]
        outs.append((rgb, op, depth))
    return outs


def _render_block(pairs_ref, w12_ref, b12_ref, wbd_ref, bs_ref, br_ref, tri_ref,
                  rgb0_ref, rgb1_ref, op0_ref, op1_ref, d0_ref, d1_ref):
    f32 = jnp.float32
    w12 = w12_ref[...]
    b12 = b12_ref[...]
    wbd = wbd_ref[...]
    tri = tri_ref[...]
    bs = bs_ref[0, 0]
    br = (br_ref[0, 0], br_ref[0, 1], br_ref[0, 2])
    s_idx = jax.lax.broadcasted_iota(jnp.int32, (1, _N_SAMPLES), 1).astype(f32)
    t_mid2 = _NEAR + (s_idx + 0.5) * _STEP

    half = _BLOCK_P // _N_CHAINS
    for i in range(_N_CHAINS):
        rows = pl.ds(i * half, half)
        pairs = pairs_ref[rows, :] * (2.0 / 3.0)
        outs = _render_pairs(pairs, w12, b12, wbd, bs, br, tri, t_mid2)
        (rgb0, op0, d0), (rgb1, op1, d1) = outs
        rgb0_ref[rows, :] = rgb0
        rgb1_ref[rows, :] = rgb1
        op0_ref[rows, :] = op0
        op1_ref[rows, :] = op1
        d0_ref[rows, :] = d0
        d1_ref[rows, :] = d1


@jax.jit
def kernel(rays, W1, b1, W_sigma, b_sigma, W_rgb, b_rgb):
    n_rays = rays.shape[0]
    n_pairs = n_rays // 2
    f32 = jnp.float32
    wcat = jnp.concatenate([W_sigma, W_rgb], axis=1).T
    z128 = jnp.zeros((4, 128), dtype=f32)
    wbd = jnp.concatenate(
        [jnp.concatenate([wcat, z128], axis=1),
         jnp.concatenate([z128, wcat], axis=1)], axis=0)
    w12 = jnp.zeros((12, 512), dtype=f32)
    w12 = w12.at[0:3, 0:128].set(W1)
    w12 = w12.at[6:9, 128:256].set(W1)
    w12 = w12.at[3:6, 256:384].set(W1)
    w12 = w12.at[9:12, 384:512].set(W1)
    b12 = jnp.concatenate([b1, b1, jnp.zeros(256, dtype=f32)]).reshape(1, 512)
    bs_2d = b_sigma.reshape(1, 1)
    br_2d = b_rgb.reshape(1, 3)
    s = _N_SAMPLES
    tri = (jnp.arange(s, dtype=jnp.int32)[:, None]
           < jnp.arange(s, dtype=jnp.int32)[None, :]).astype(f32)
    pairs = rays.reshape(n_pairs, 12)
    grid = (n_pairs // _BLOCK_P,)
    out2 = [
        jax.ShapeDtypeStruct((n_pairs, 3), f32),
        jax.ShapeDtypeStruct((n_pairs, 3), f32),
        jax.ShapeDtypeStruct((n_pairs, 1), f32),
        jax.ShapeDtypeStruct((n_pairs, 1), f32),
        jax.ShapeDtypeStruct((n_pairs, 1), f32),
        jax.ShapeDtypeStruct((n_pairs, 1), f32),
    ]
    rgb0, rgb1, op0, op1, d0, d1 = pl.pallas_call(
        _render_block,
        grid=grid,
        in_specs=[
            pl.BlockSpec((_BLOCK_P, 12), lambda i: (i, 0)),
            pl.BlockSpec((12, 512), lambda i: (0, 0)),
            pl.BlockSpec((1, 512), lambda i: (0, 0)),
            pl.BlockSpec((8, 256), lambda i: (0, 0)),
            pl.BlockSpec((1, 1), lambda i: (0, 0)),
            pl.BlockSpec((1, 3), lambda i: (0, 0)),
            pl.BlockSpec((s, s), lambda i: (0, 0)),
        ],
        out_specs=[
            pl.BlockSpec((_BLOCK_P, 3), lambda i: (i, 0)),
            pl.BlockSpec((_BLOCK_P, 3), lambda i: (i, 0)),
            pl.BlockSpec((_BLOCK_P, 1), lambda i: (i, 0)),
            pl.BlockSpec((_BLOCK_P, 1), lambda i: (i, 0)),
            pl.BlockSpec((_BLOCK_P, 1), lambda i: (i, 0)),
            pl.BlockSpec((_BLOCK_P, 1), lambda i: (i, 0)),
        ],
        out_shape=out2,
    )(pairs, w12, b12, wbd, bs_2d, br_2d, tri)
    rgb = jnp.stack([rgb0, rgb1], axis=1).reshape(n_rays, 3)
    op = jnp.stack([op0[:, 0], op1[:, 0]], axis=1).reshape(n_rays)
    depth = jnp.stack([d0[:, 0], d1[:, 0]], axis=1).reshape(n_rays)
    return rgb, op, depth


# speedup vs baseline: 1.3837x; 1.0266x over previous
import jax
import jax.numpy as jnp
from jax.experimental import pallas as pl

_N_RAYS = 4096
_N_SAMPLES = 192
_NEAR, _FAR = 2.0, 6.0
_STEP = (_FAR - _NEAR) / _N_SAMPLES
_LOG_EPS = -23.025850929940457
_BLOCK_R = 512
_N_CHAINS = 2


def _render_rays(rays, w6, b6, wcat, bs, br, tri, t_mid2):
    f32 = jnp.float32
    r = rays.shape[0]
    ab = jnp.dot(rays, w6, preferred_element_type=f32) + b6
    c = ab.reshape(r, 2, 128)
    ones = jnp.ones((1, _N_SAMPLES), dtype=f32)
    tmat = jnp.concatenate([ones[:, None, :], t_mid2[:, None, :]], axis=1)
    tmat = jnp.broadcast_to(tmat, (r, 2, _N_SAMPLES))
    h = jax.lax.dot_general(
        c, tmat, dimension_numbers=(((1,), (1,)), ((0,), (0,))),
        preferred_element_type=f32)
    h = jnp.maximum(h, 0.0)

    wcat_b = jnp.broadcast_to(wcat[None], (r, 4, 128))
    z = jax.lax.dot_general(
        wcat_b, h, dimension_numbers=(((2,), (1,)), ((0,), (0,))),
        preferred_element_type=f32)

    sigma = jax.nn.softplus(z[:, 0, :] + bs)
    x = sigma * _STEP
    alpha = 1.0 - jnp.exp(-x)
    log_trans = jnp.maximum(-x, _LOG_EPS)
    excl = jnp.dot(log_trans, tri, preferred_element_type=f32)
    weights = alpha * jnp.exp(excl)

    outs = []
    for ch in range(3):
        rgb_c = jax.nn.sigmoid(z[:, 1 + ch, :] + br[ch])
        outs.append(jnp.sum(weights * rgb_c, axis=-1)[:, None])
    rgb = jnp.concatenate(outs, axis=1)
    op = jnp.sum(weights, axis=-1)[:, None]
    depth = jnp.sum(weights * t_mid2, axis=-1)[:, None]
    return rgb, op, depth


def _render_block(rays_ref, w1_ref, b1_ref, wcat_ref, bs_ref, br_ref,
                  rgb_ref, op_ref, depth_ref):
    f32 = jnp.float32
    w1 = w1_ref[...]
    z3 = jnp.zeros((3, 128), dtype=f32)
    w6 = jnp.concatenate(
        [jnp.concatenate([w1, z3], axis=1),
         jnp.concatenate([z3, w1], axis=1)], axis=0)
    b1 = b1_ref[...]
    b6 = jnp.concatenate([b1, jnp.zeros_like(b1)], axis=1)
    wcat = wcat_ref[...]
    bs = bs_ref[0, 0]
    br = (br_ref[0, 0], br_ref[0, 1], br_ref[0, 2])
    s = _N_SAMPLES
    row = jax.lax.broadcasted_iota(jnp.int32, (s, s), 0)
    col = jax.lax.broadcasted_iota(jnp.int32, (s, s), 1)
    tri = (row < col).astype(f32)
    s_idx = jax.lax.broadcasted_iota(jnp.int32, (1, s), 1).astype(f32)
    t_mid2 = _NEAR + (s_idx + 0.5) * _STEP

    half = _BLOCK_R // _N_CHAINS
    for i in range(_N_CHAINS):
        rows = pl.ds(i * half, half)
        rays = rays_ref[rows, :] * (2.0 / 3.0)
        rgb, op, depth = _render_rays(rays, w6, b6, wcat, bs, br, tri, t_mid2)
        rgb_ref[rows, :] = rgb
        op_ref[rows, :] = op
        depth_ref[rows, :] = depth


@jax.jit
def kernel(rays, W1, b1, W_sigma, b_sigma, W_rgb, b_rgb):
    n_rays = rays.shape[0]
    wcat = jnp.concatenate([W_sigma, W_rgb], axis=1).T
    b1_2d = b1.reshape(1, 128)
    bs_2d = b_sigma.reshape(1, 1)
    br_2d = b_rgb.reshape(1, 3)
    grid = (n_rays // _BLOCK_R,)
    rgb, op, depth = pl.pallas_call(
        _render_block,
        grid=grid,
        in_specs=[
            pl.BlockSpec((_BLOCK_R, 6), lambda i: (i, 0)),
            pl.BlockSpec((3, 128), lambda i: (0, 0)),
            pl.BlockSpec((1, 128), lambda i: (0, 0)),
            pl.BlockSpec((4, 128), lambda i: (0, 0)),
            pl.BlockSpec((1, 1), lambda i: (0, 0)),
            pl.BlockSpec((1, 3), lambda i: (0, 0)),
        ],
        out_specs=[
            pl.BlockSpec((_BLOCK_R, 3), lambda i: (i, 0)),
            pl.BlockSpec((_BLOCK_R, 1), lambda i: (i, 0)),
            pl.BlockSpec((_BLOCK_R, 1), lambda i: (i, 0)),
        ],
        out_shape=[
            jax.ShapeDtypeStruct((n_rays, 3), jnp.float32),
            jax.ShapeDtypeStruct((n_rays, 1), jnp.float32),
            jax.ShapeDtypeStruct((n_rays, 1), jnp.float32),
        ],
    )(rays, W1, b1_2d, wcat, bs_2d, br_2d)
    return rgb, op[:, 0], depth[:, 0]
